# baseline (speedup 1.0000x reference)
import dataclasses
import functools

import jax
import jax.numpy as jnp
from jax import lax
from jax.experimental import pallas as pl
from jax.experimental.pallas import tpu as pltpu
from jax.experimental.pallas import tpu_sc as plsc

N = 10000
E = 160000
F = 256
T = 4
D = 64
NW = 32
OWN = 128
SLOTS = 80
NP = OWN * SLOTS
EPW = E // NW
GW = 128
SEG = 4000
PREC = lax.Precision.HIGHEST


def _sc_params():
    cp = pltpu.CompilerParams()
    if "needs_layout_passes" in pltpu.CompilerParams.__dataclass_fields__:
        cp = dataclasses.replace(cp, needs_layout_passes=False)
    return cp


def _dot(a, b):
    return jnp.dot(a, b, preferred_element_type=jnp.float32, precision=PREC)


def _pre_body(x_ref, wd_ref, ws_ref, b1_ref, a_ref, b_ref):
    x = x_ref[...]
    for t in range(T):
        xt = x[:, t * D:(t + 1) * D]
        a_ref[:, t * D:(t + 1) * D] = _dot(xt, wd_ref[t]) + b1_ref[t][None, :]
        b_ref[:, t * D:(t + 1) * D] = _dot(xt, ws_ref[t])


def _pre(x, wd, ws, b1):
    blk = 1000
    return pl.pallas_call(
        _pre_body,
        grid=(N // blk,),
        in_specs=[
            pl.BlockSpec((blk, F), lambda i: (i, 0)),
            pl.BlockSpec((T, D, D), lambda i: (0, 0, 0)),
            pl.BlockSpec((T, D, D), lambda i: (0, 0, 0)),
            pl.BlockSpec((T, D), lambda i: (0, 0)),
        ],
        out_specs=[
            pl.BlockSpec((blk, F), lambda i: (i, 0)),
            pl.BlockSpec((blk, F), lambda i: (i, 0)),
        ],
        out_shape=[
            jax.ShapeDtypeStruct((N, F), jnp.float32),
            jax.ShapeDtypeStruct((N, F), jnp.float32),
        ],
    )(x, wd, ws, b1)


def _gather_body(a_hbm, b_hbm, dst_hbm, src_hbm, u_hbm,
                 didx, sidx, bufa, bufb, sema, semb):
    wid = lax.axis_index("s") * 2 + lax.axis_index("c")
    base = wid * EPW
    col = [jnp.arange(16, dtype=jnp.int32) + 16 * k for k in range(F // 16)]

    def add_rows(nrows):
        @pl.loop(0, nrows)
        def _(r):
            rb = jnp.broadcast_to(r, (16,)).astype(jnp.int32)
            for k in range(F // 16):
                va = plsc.load_gather(bufa, [rb, col[k]])
                vb = plsc.load_gather(bufb, [rb, col[k]])
                plsc.store_scatter(bufa, [rb, col[k]], va + vb)

    @pl.loop(0, EPW // GW)
    def _(w):
        wb = base + w * GW
        pltpu.sync_copy(dst_hbm.at[pl.ds(wb, GW)], didx)
        pltpu.sync_copy(src_hbm.at[pl.ds(wb, GW)], sidx)
        ca = pltpu.async_copy(a_hbm.at[didx], bufa, sema)
        cb = pltpu.async_copy(b_hbm.at[sidx], bufb, semb)
        ca.wait()
        cb.wait()
        add_rows(GW)
        pltpu.sync_copy(bufa, u_hbm.at[pl.ds(wb, GW)])

    tail = EPW - (EPW // GW) * GW
    if tail:
        wb = base + (EPW // GW) * GW
        pltpu.sync_copy(dst_hbm.at[pl.ds(wb, tail)], didx.at[pl.ds(0, tail)])
        pltpu.sync_copy(src_hbm.at[pl.ds(wb, tail)], sidx.at[pl.ds(0, tail)])
        ca = pltpu.async_copy(a_hbm.at[didx.at[pl.ds(0, tail)]],
                              bufa.at[pl.ds(0, tail)], sema)
        cb = pltpu.async_copy(b_hbm.at[sidx.at[pl.ds(0, tail)]],
                              bufb.at[pl.ds(0, tail)], semb)
        ca.wait()
        cb.wait()
        add_rows(tail)
        pltpu.sync_copy(bufa.at[pl.ds(0, tail)], u_hbm.at[pl.ds(wb, tail)])


def _gather_add(a, b, dst, src):
    mesh = plsc.VectorSubcoreMesh(core_axis_name="c", subcore_axis_name="s")
    return pl.kernel(
        _gather_body,
        out_type=jax.ShapeDtypeStruct((E, F), jnp.float32),
        mesh=mesh,
        compiler_params=_sc_params(),
        scratch_types=[
            pltpu.VMEM((GW,), jnp.int32),
            pltpu.VMEM((GW,), jnp.int32),
            pltpu.VMEM((GW, F), jnp.float32),
            pltpu.VMEM((GW, F), jnp.float32),
            pltpu.SemaphoreType.DMA,
            pltpu.SemaphoreType.DMA,
        ],
    )(a, b, dst, src)


def _mid_body(u_ref, w2_ref, b2_ref, h_ref):
    u = u_ref[...]
    for t in range(T):
        r = jnp.maximum(u[:, t * D:(t + 1) * D], 0.0)
        h_ref[:, t * D:(t + 1) * D] = _dot(r, w2_ref[t]) + b2_ref[t][None, :]


def _edge_mlp(u, w2, b2):
    blk = 2000
    return pl.pallas_call(
        _mid_body,
        grid=(E // blk,),
        in_specs=[
            pl.BlockSpec((blk, F), lambda i: (i, 0)),
            pl.BlockSpec((T, D, D), lambda i: (0, 0, 0)),
            pl.BlockSpec((T, D), lambda i: (0, 0)),
        ],
        out_specs=pl.BlockSpec((blk, F), lambda i: (i, 0)),
        out_shape=jax.ShapeDtypeStruct((E, F), jnp.float32),
    )(u, w2, b2)


def _reduce_body(dst_hbm, h_hbm, sum_o, min_o, max_o, deg_o,
                 dbuf, elist, jlist, hbuf, oidx, accs, accn, accx, accd):
    wid = (lax.axis_index("s") * 2 + lax.axis_index("c")).astype(jnp.int32)
    iota = jnp.arange(16, dtype=jnp.int32)
    col = [iota + 16 * k for k in range(F // 16)]
    zeros = jnp.zeros((16,), jnp.float32)
    ones = jnp.ones((16,), jnp.float32)
    pinf = jnp.full((16,), jnp.inf, jnp.float32)
    ninf = jnp.full((16,), -jnp.inf, jnp.float32)
    truem = jnp.full((16,), True)

    @pl.loop(0, NP // (NW * SLOTS))
    def _(r):
        owner = wid + NW * r

        zb = jnp.broadcast_to(jnp.int32(0), (16,))
        for w in range(SLOTS // 16):
            plsc.store_scatter(oidx, [zb, 16 * w + iota],
                               owner + OWN * (16 * w + iota))

        @pl.loop(0, SLOTS)
        def _(q):
            qb = jnp.broadcast_to(q, (16,)).astype(jnp.int32)
            for k in range(F // 16):
                plsc.store_scatter(accs, [qb, col[k]], zeros)
                plsc.store_scatter(accn, [qb, col[k]], pinf)
                plsc.store_scatter(accx, [qb, col[k]], ninf)
            for k in range(8):
                plsc.store_scatter(accd, [qb, col[k]], zeros)

        @pl.loop(0, E // SEG)
        def _(s):
            segb = s * SEG
            pltpu.sync_copy(dst_hbm.at[pl.ds(segb, SEG)], dbuf)

            @pl.loop(0, SEG // 16, init_carry=jnp.int32(0))
            def scan(v, cnt):
                d = dbuf[pl.ds(16 * v, 16)]
                m = (d & (OWN - 1)) == owner
                j = lax.shift_right_logical(d, 7)
                eid = segb + 16 * v + iota
                plsc.store_compressed(elist.at[pl.ds(cnt, 16)], eid, mask=m)
                plsc.store_compressed(jlist.at[pl.ds(cnt, 16)], j, mask=m)
                return cnt + jnp.sum(m.astype(jnp.int32))

            cnt = scan
            for w in range(8):
                plsc.store_compressed(
                    elist.at[pl.ds(cnt + 16 * w, 16)],
                    jnp.zeros((16,), jnp.int32), mask=truem)
                plsc.store_compressed(
                    jlist.at[pl.ds(cnt + 16 * w, 16)],
                    jnp.full((16,), SLOTS - 1, jnp.int32), mask=truem)

            @pl.loop(0, (cnt + 127) // 128)
            def _(c):
                cb = c * 128
                pltpu.sync_copy(h_hbm.at[elist.at[pl.ds(cb, 128)]], hbuf)

                @pl.loop(0, 8)
                def _(v):
                    jv = jlist[pl.ds(cb + 16 * v, 16)]
                    for l in range(16):
                        rb = jnp.broadcast_to(16 * v + l,
                                              (16,)).astype(jnp.int32)
                        jb = jnp.broadcast_to(jv[l], (16,))
                        for k in range(F // 16):
                            hv = plsc.load_gather(hbuf, [rb, col[k]])
                            plsc.addupdate_scatter(accs, [jb, col[k]], hv)
                            cn = plsc.load_gather(accn, [jb, col[k]])
                            plsc.store_scatter(accn, [jb, col[k]],
                                               jnp.minimum(cn, hv))
                            cx = plsc.load_gather(accx, [jb, col[k]])
                            plsc.store_scatter(accx, [jb, col[k]],
                                               jnp.maximum(cx, hv))
                        plsc.addupdate_scatter(accd, [jb, iota], ones)

        pltpu.sync_copy(accs, sum_o.at[oidx.at[0]])
        pltpu.sync_copy(accn, min_o.at[oidx.at[0]])
        pltpu.sync_copy(accx, max_o.at[oidx.at[0]])
        pltpu.sync_copy(accd, deg_o.at[oidx.at[0]])


def _segment_reduce(dst, h):
    mesh = plsc.VectorSubcoreMesh(core_axis_name="c", subcore_axis_name="s")
    return pl.kernel(
        _reduce_body,
        out_type=[
            jax.ShapeDtypeStruct((NP, F), jnp.float32),
            jax.ShapeDtypeStruct((NP, F), jnp.float32),
            jax.ShapeDtypeStruct((NP, F), jnp.float32),
            jax.ShapeDtypeStruct((NP, 128), jnp.float32),
        ],
        mesh=mesh,
        compiler_params=_sc_params(),
        scratch_types=[
            pltpu.VMEM((SEG,), jnp.int32),
            pltpu.VMEM((SEG + 224,), jnp.int32),
            pltpu.VMEM((SEG + 224,), jnp.int32),
            pltpu.VMEM((128, F), jnp.float32),
            pltpu.VMEM((1, SLOTS), jnp.int32),
            pltpu.VMEM((SLOTS, F), jnp.float32),
            pltpu.VMEM((SLOTS, F), jnp.float32),
            pltpu.VMEM((SLOTS, F), jnp.float32),
            pltpu.VMEM((SLOTS, 128), jnp.float32),
        ],
    )(dst, h)


def _post_body(x_ref, s_ref, n_ref, m_ref, dg_ref,
               p0x_ref, p0a_ref, pb0_ref, p1_ref, pb1_ref, wm_ref, bm_ref,
               o_ref):
    inv_log = 1.0 / jnp.log(jnp.float32(17.0))
    deg = dg_ref[:, 0:1]
    deg_safe = jnp.maximum(deg, 1.0)
    amp = jnp.log(deg + 1.0) * inv_log
    he = deg > 0.0
    x = x_ref[...]
    ys = []
    for t in range(T):
        xt = x[:, t * D:(t + 1) * D]
        s = s_ref[:, t * D:(t + 1) * D]
        mn = jnp.where(he, n_ref[:, t * D:(t + 1) * D], 0.0)
        mx = jnp.where(he, m_ref[:, t * D:(t + 1) * D], 0.0)
        mean = s / deg_safe
        cat = jnp.concatenate(
            [mean, mean * amp, mx, mx * amp, mn, mn * amp, s, s * amp], axis=1)
        o = jnp.maximum(_dot(xt, p0x_ref[t]) + _dot(cat, p0a_ref[t])
                        + pb0_ref[t][None, :], 0.0)
        ys.append(_dot(o, p1_ref[t]) + pb1_ref[t][None, :])
    y = jnp.concatenate(ys, axis=1)
    z = _dot(y, wm_ref[...]) + bm_ref[...][None, :]
    o_ref[...] = jnp.where(z > 0.0, z, 0.01 * z)


def _post(x, s4, n4, m4, degb, p0x, p0a, pb0, p1, pb1, wm, bm):
    blk = 1000
    return pl.pallas_call(
        _post_body,
        grid=(N // blk,),
        in_specs=[
            pl.BlockSpec((blk, F), lambda i: (i, 0)),
            pl.BlockSpec((blk, F), lambda i: (i, 0)),
            pl.BlockSpec((blk, F), lambda i: (i, 0)),
            pl.BlockSpec((blk, F), lambda i: (i, 0)),
            pl.BlockSpec((blk, 128), lambda i: (i, 0)),
            pl.BlockSpec((T, D, D), lambda i: (0, 0, 0)),
            pl.BlockSpec((T, 8 * D, D), lambda i: (0, 0, 0)),
            pl.BlockSpec((T, D), lambda i: (0, 0)),
            pl.BlockSpec((T, D, D), lambda i: (0, 0, 0)),
            pl.BlockSpec((T, D), lambda i: (0, 0)),
            pl.BlockSpec((F, F), lambda i: (0, 0)),
            pl.BlockSpec((F,), lambda i: (0,)),
        ],
        out_specs=pl.BlockSpec((blk, F), lambda i: (i, 0)),
        out_shape=jax.ShapeDtypeStruct((N, F), jnp.float32),
    )(x, s4, n4, m4, degb, p0x, p0a, pb0, p1, pb1, wm, bm)


def kernel(x, edge_index, params):
    src = edge_index[0].astype(jnp.int32)
    dst = edge_index[1].astype(jnp.int32)
    towers = params["towers"]
    wd = jnp.stack([tw["pre"][0]["W"][:D] for tw in towers])
    ws = jnp.stack([tw["pre"][0]["W"][D:] for tw in towers])
    b1 = jnp.stack([tw["pre"][0]["b"] for tw in towers])
    w2 = jnp.stack([tw["pre"][1]["W"] for tw in towers])
    b2 = jnp.stack([tw["pre"][1]["b"] for tw in towers])
    p0x = jnp.stack([tw["post"][0]["W"][:D] for tw in towers])
    p0a = jnp.stack([tw["post"][0]["W"][D:] for tw in towers])
    pb0 = jnp.stack([tw["post"][0]["b"] for tw in towers])
    p1 = jnp.stack([tw["post"][1]["W"] for tw in towers])
    pb1 = jnp.stack([tw["post"][1]["b"] for tw in towers])
    wm = params["mix"]["W"]
    bm = params["mix"]["b"]

    a, b = _pre(x, wd, ws, b1)
    u = _gather_add(a, b, dst, src)
    h = _edge_mlp(u, w2, b2)
    s4, n4, m4, degb = _segment_reduce(dst, h)
    return _post(x, s4, n4, m4, degb, p0x, p0a, pb0, p1, pb1, wm, bm)


# speedup vs baseline: 2.7456x; 2.7456x over previous
import dataclasses
import functools

import jax
import jax.numpy as jnp
from jax import lax
from jax.experimental import pallas as pl
from jax.experimental.pallas import tpu as pltpu
from jax.experimental.pallas import tpu_sc as plsc

N = 10000
E = 160000
F = 256
T = 4
D = 64
NW = 32
OWN = 128
SLOTS = 80
NP = OWN * SLOTS
EPW = E // NW
GW = 128
SEG = 4000
PREC = lax.Precision.HIGHEST


def _sc_params():
    cp = pltpu.CompilerParams()
    if "needs_layout_passes" in pltpu.CompilerParams.__dataclass_fields__:
        cp = dataclasses.replace(cp, needs_layout_passes=False)
    return cp


def _dot(a, b):
    return jnp.dot(a, b, preferred_element_type=jnp.float32, precision=PREC)


def _pre_body(x_ref, wd_ref, ws_ref, b1_ref, a_ref, b_ref):
    x = x_ref[...]
    for t in range(T):
        xt = x[:, t * D:(t + 1) * D]
        a_ref[:, t * D:(t + 1) * D] = _dot(xt, wd_ref[t]) + b1_ref[t][None, :]
        b_ref[:, t * D:(t + 1) * D] = _dot(xt, ws_ref[t])


def _pre(x, wd, ws, b1):
    blk = 1000
    return pl.pallas_call(
        _pre_body,
        grid=(N // blk,),
        in_specs=[
            pl.BlockSpec((blk, F), lambda i: (i, 0)),
            pl.BlockSpec((T, D, D), lambda i: (0, 0, 0)),
            pl.BlockSpec((T, D, D), lambda i: (0, 0, 0)),
            pl.BlockSpec((T, D), lambda i: (0, 0)),
        ],
        out_specs=[
            pl.BlockSpec((blk, F), lambda i: (i, 0)),
            pl.BlockSpec((blk, F), lambda i: (i, 0)),
        ],
        out_shape=[
            jax.ShapeDtypeStruct((N, F), jnp.float32),
            jax.ShapeDtypeStruct((N, F), jnp.float32),
        ],
    )(x, wd, ws, b1)


GW2 = 64
NWIN2 = E // GW2


def _gather_body(a_hbm, b_hbm, dst_hbm, src_hbm, u_hbm,
                 didx, sidx, bufa, bufb, sa0, sa1, sb0, sb1):
    wid = lax.axis_index("s") * 2 + lax.axis_index("c")
    nmine = (NWIN2 - 1 - wid) // NW + 1
    col = [jnp.arange(16, dtype=jnp.int32) + 16 * k for k in range(F // 16)]
    sems = [(sa0, sb0), (sa1, sb1)]

    def win_base(i):
        return (wid + NW * i) * GW2

    def issue(i, ph):
        wb = win_base(i)
        pltpu.sync_copy(dst_hbm.at[pl.ds(wb, GW2)], didx.at[ph])
        pltpu.sync_copy(src_hbm.at[pl.ds(wb, GW2)], sidx.at[ph])
        pltpu.async_copy(a_hbm.at[didx.at[ph]], bufa.at[ph], sems[ph][0])
        pltpu.async_copy(b_hbm.at[sidx.at[ph]], bufb.at[ph], sems[ph][1])

    def wait_gathers(ph):
        pltpu.make_async_copy(a_hbm.at[didx.at[ph]], bufa.at[ph],
                              sems[ph][0]).wait()
        pltpu.make_async_copy(b_hbm.at[sidx.at[ph]], bufb.at[ph],
                              sems[ph][1]).wait()

    def add_rows(ph):
        @pl.loop(0, GW2)
        def _(r):
            rb = jnp.broadcast_to(r, (16,)).astype(jnp.int32)
            for k in range(F // 16):
                va = plsc.load_gather(bufa.at[ph], [rb, col[k]])
                vb = plsc.load_gather(bufb.at[ph], [rb, col[k]])
                plsc.store_scatter(bufa.at[ph], [rb, col[k]], va + vb)

    issue(0, 0)

    @pl.loop(0, (nmine + 1) // 2)
    def _(pair):
        for ph in range(2):
            i = 2 * pair + ph

            @pl.when(i < nmine)
            def _():
                wait_gathers(ph)

                @pl.when(i + 1 < nmine)
                def _():
                    issue(i + 1, 1 - ph)

                add_rows(ph)
                pltpu.sync_copy(bufa.at[ph],
                                u_hbm.at[pl.ds(win_base(i), GW2)])


def _gather_add(a, b, dst, src):
    mesh = plsc.VectorSubcoreMesh(core_axis_name="c", subcore_axis_name="s")
    return pl.kernel(
        _gather_body,
        out_type=jax.ShapeDtypeStruct((E, F), jnp.float32),
        mesh=mesh,
        compiler_params=_sc_params(),
        scratch_types=[
            pltpu.VMEM((2, GW2), jnp.int32),
            pltpu.VMEM((2, GW2), jnp.int32),
            pltpu.VMEM((2, GW2, F), jnp.float32),
            pltpu.VMEM((2, GW2, F), jnp.float32),
            pltpu.SemaphoreType.DMA,
            pltpu.SemaphoreType.DMA,
            pltpu.SemaphoreType.DMA,
            pltpu.SemaphoreType.DMA,
        ],
    )(a, b, dst, src)


def _mid_body(u_ref, w2_ref, b2_ref, h_ref):
    u = u_ref[...]
    for t in range(T):
        r = jnp.maximum(u[:, t * D:(t + 1) * D], 0.0)
        h_ref[:, t * D:(t + 1) * D] = _dot(r, w2_ref[t]) + b2_ref[t][None, :]


def _edge_mlp(u, w2, b2):
    blk = 2000
    return pl.pallas_call(
        _mid_body,
        grid=(E // blk,),
        in_specs=[
            pl.BlockSpec((blk, F), lambda i: (i, 0)),
            pl.BlockSpec((T, D, D), lambda i: (0, 0, 0)),
            pl.BlockSpec((T, D), lambda i: (0, 0)),
        ],
        out_specs=pl.BlockSpec((blk, F), lambda i: (i, 0)),
        out_shape=jax.ShapeDtypeStruct((E, F), jnp.float32),
    )(u, w2, b2)


def _reduce_body(dst_hbm, h_hbm, sum_o, min_o, max_o, deg_o,
                 dbuf, elist, jlist, hbuf, oidx, accs, accn, accx, accd):
    wid = (lax.axis_index("s") * 2 + lax.axis_index("c")).astype(jnp.int32)
    iota = jnp.arange(16, dtype=jnp.int32)
    col = [iota + 16 * k for k in range(F // 16)]
    zeros = jnp.zeros((16,), jnp.float32)
    ones = jnp.ones((16,), jnp.float32)
    pinf = jnp.full((16,), jnp.inf, jnp.float32)
    ninf = jnp.full((16,), -jnp.inf, jnp.float32)
    truem = jnp.full((16,), True)

    @pl.loop(0, NP // (NW * SLOTS))
    def _(r):
        owner = wid + NW * r

        zb = jnp.broadcast_to(jnp.int32(0), (16,))
        for w in range(SLOTS // 16):
            plsc.store_scatter(oidx, [zb, 16 * w + iota],
                               owner + OWN * (16 * w + iota))

        @pl.loop(0, SLOTS)
        def _(q):
            qb = jnp.broadcast_to(q, (16,)).astype(jnp.int32)
            for k in range(F // 16):
                plsc.store_scatter(accs, [qb, col[k]], zeros)
                plsc.store_scatter(accn, [qb, col[k]], pinf)
                plsc.store_scatter(accx, [qb, col[k]], ninf)
            for k in range(8):
                plsc.store_scatter(accd, [qb, col[k]], zeros)

        @pl.loop(0, E // SEG)
        def _(s):
            segb = s * SEG
            pltpu.sync_copy(dst_hbm.at[pl.ds(segb, SEG)], dbuf)

            @pl.loop(0, SEG // 16, init_carry=jnp.int32(0))
            def scan(v, cnt):
                d = dbuf[pl.ds(16 * v, 16)]
                m = (d & (OWN - 1)) == owner
                j = lax.shift_right_logical(d, 7)
                eid = segb + 16 * v + iota
                plsc.store_compressed(elist.at[pl.ds(cnt, 16)], eid, mask=m)
                plsc.store_compressed(jlist.at[pl.ds(cnt, 16)], j, mask=m)
                return cnt + plsc.all_reduce_population_count(m)[0]

            cnt = scan
            for w in range(2):
                plsc.store_compressed(
                    elist.at[pl.ds(cnt + 16 * w, 16)],
                    segb + 16 * w + iota, mask=truem)
                plsc.store_compressed(
                    jlist.at[pl.ds(cnt + 16 * w, 16)],
                    jnp.full((16,), SLOTS - 1, jnp.int32), mask=truem)

            ngroups = (cnt + 15) // 16

            @pl.loop(0, (cnt + 31) // 32)
            def _(c):
                cb = c * 32
                pltpu.sync_copy(h_hbm.at[elist.at[pl.ds(cb, 32)]], hbuf)

                @pl.loop(0, jnp.minimum(2, ngroups - 2 * c))
                def _(g):
                    gb = cb + 16 * g
                    jv = jlist[pl.ds(gb, 16)]
                    js, lane_s = plsc.sort_key_val(jv, iota)
                    prev = js[jnp.maximum(iota - 1, 0)]
                    head = (js != prev) | (iota == 0)
                    runpos = plsc.cummax(jnp.where(head, iota, 0))
                    occ = iota - runpos
                    valid = (gb + lane_s) < cnt
                    occv = jnp.where(valid, occ, 0)
                    rowv = 16 * g + lane_s

                    @pl.loop(0, jnp.max(occv) + 1)
                    def _(k):
                        mk = valid & (occ == k)
                        plsc.addupdate_scatter(accd, [js, iota], ones,
                                               mask=mk)

                        @pl.loop(0, F // 8)
                        def _(o):
                            for kk in range(8):
                                cful = jnp.broadcast_to(
                                    8 * o + kk, (16,)).astype(jnp.int32)
                                hv = plsc.load_gather(hbuf, [rowv, cful],
                                                      mask=mk)
                                plsc.addupdate_scatter(accs, [js, cful],
                                                       hv, mask=mk)
                                cn = plsc.load_gather(accn, [js, cful],
                                                      mask=mk)
                                plsc.store_scatter(accn, [js, cful],
                                                   jnp.minimum(cn, hv),
                                                   mask=mk)
                                cx = plsc.load_gather(accx, [js, cful],
                                                      mask=mk)
                                plsc.store_scatter(accx, [js, cful],
                                                   jnp.maximum(cx, hv),
                                                   mask=mk)

        pltpu.sync_copy(accs, sum_o.at[oidx.at[0]])
        pltpu.sync_copy(accn, min_o.at[oidx.at[0]])
        pltpu.sync_copy(accx, max_o.at[oidx.at[0]])
        pltpu.sync_copy(accd, deg_o.at[oidx.at[0]])


def _segment_reduce(dst, h):
    mesh = plsc.VectorSubcoreMesh(core_axis_name="c", subcore_axis_name="s")
    return pl.kernel(
        _reduce_body,
        out_type=[
            jax.ShapeDtypeStruct((NP, F), jnp.float32),
            jax.ShapeDtypeStruct((NP, F), jnp.float32),
            jax.ShapeDtypeStruct((NP, F), jnp.float32),
            jax.ShapeDtypeStruct((NP, 128), jnp.float32),
        ],
        mesh=mesh,
        compiler_params=_sc_params(),
        scratch_types=[
            pltpu.VMEM((SEG,), jnp.int32),
            pltpu.VMEM((SEG + 48,), jnp.int32),
            pltpu.VMEM((SEG + 48,), jnp.int32),
            pltpu.VMEM((32, F), jnp.float32),
            pltpu.VMEM((1, SLOTS), jnp.int32),
            pltpu.VMEM((SLOTS, F), jnp.float32),
            pltpu.VMEM((SLOTS, F), jnp.float32),
            pltpu.VMEM((SLOTS, F), jnp.float32),
            pltpu.VMEM((SLOTS, 128), jnp.float32),
        ],
    )(dst, h)


def _post_body(x_ref, s_ref, n_ref, m_ref, dg_ref,
               p0x_ref, p0a_ref, pb0_ref, p1_ref, pb1_ref, wm_ref, bm_ref,
               o_ref):
    inv_log = 1.0 / jnp.log(jnp.float32(17.0))
    deg = jnp.sum(dg_ref[...], axis=1, keepdims=True)
    deg_safe = jnp.maximum(deg, 1.0)
    amp = jnp.log(deg + 1.0) * inv_log
    he = deg > 0.0
    x = x_ref[...]
    ys = []
    for t in range(T):
        xt = x[:, t * D:(t + 1) * D]
        s = s_ref[:, t * D:(t + 1) * D]
        mn = jnp.where(he, n_ref[:, t * D:(t + 1) * D], 0.0)
        mx = jnp.where(he, m_ref[:, t * D:(t + 1) * D], 0.0)
        mean = s / deg_safe
        cat = jnp.concatenate(
            [mean, mean * amp, mx, mx * amp, mn, mn * amp, s, s * amp], axis=1)
        o = jnp.maximum(_dot(xt, p0x_ref[t]) + _dot(cat, p0a_ref[t])
                        + pb0_ref[t][None, :], 0.0)
        ys.append(_dot(o, p1_ref[t]) + pb1_ref[t][None, :])
    y = jnp.concatenate(ys, axis=1)
    z = _dot(y, wm_ref[...]) + bm_ref[...][None, :]
    o_ref[...] = jnp.where(z > 0.0, z, 0.01 * z)


def _post(x, s4, n4, m4, degb, p0x, p0a, pb0, p1, pb1, wm, bm):
    blk = 1000
    return pl.pallas_call(
        _post_body,
        grid=(N // blk,),
        in_specs=[
            pl.BlockSpec((blk, F), lambda i: (i, 0)),
            pl.BlockSpec((blk, F), lambda i: (i, 0)),
            pl.BlockSpec((blk, F), lambda i: (i, 0)),
            pl.BlockSpec((blk, F), lambda i: (i, 0)),
            pl.BlockSpec((blk, 128), lambda i: (i, 0)),
            pl.BlockSpec((T, D, D), lambda i: (0, 0, 0)),
            pl.BlockSpec((T, 8 * D, D), lambda i: (0, 0, 0)),
            pl.BlockSpec((T, D), lambda i: (0, 0)),
            pl.BlockSpec((T, D, D), lambda i: (0, 0, 0)),
            pl.BlockSpec((T, D), lambda i: (0, 0)),
            pl.BlockSpec((F, F), lambda i: (0, 0)),
            pl.BlockSpec((F,), lambda i: (0,)),
        ],
        out_specs=pl.BlockSpec((blk, F), lambda i: (i, 0)),
        out_shape=jax.ShapeDtypeStruct((N, F), jnp.float32),
    )(x, s4, n4, m4, degb, p0x, p0a, pb0, p1, pb1, wm, bm)


def kernel(x, edge_index, params):
    src = edge_index[0].astype(jnp.int32)
    dst = edge_index[1].astype(jnp.int32)
    towers = params["towers"]
    wd = jnp.stack([tw["pre"][0]["W"][:D] for tw in towers])
    ws = jnp.stack([tw["pre"][0]["W"][D:] for tw in towers])
    b1 = jnp.stack([tw["pre"][0]["b"] for tw in towers])
    w2 = jnp.stack([tw["pre"][1]["W"] for tw in towers])
    b2 = jnp.stack([tw["pre"][1]["b"] for tw in towers])
    p0x = jnp.stack([tw["post"][0]["W"][:D] for tw in towers])
    p0a = jnp.stack([tw["post"][0]["W"][D:] for tw in towers])
    pb0 = jnp.stack([tw["post"][0]["b"] for tw in towers])
    p1 = jnp.stack([tw["post"][1]["W"] for tw in towers])
    pb1 = jnp.stack([tw["post"][1]["b"] for tw in towers])
    wm = params["mix"]["W"]
    bm = params["mix"]["b"]

    a, b = _pre(x, wd, ws, b1)
    u = _gather_add(a, b, dst, src)
    h = _edge_mlp(u, w2, b2)
    s4, n4, m4, degb = _segment_reduce(dst, h)
    return _post(x, s4, n4, m4, degb, p0x, p0a, pb0, p1, pb1, wm, bm)
